# baseline (speedup 1.0000x reference)
import functools

import jax
import jax.numpy as jnp
from jax import lax
from jax.experimental import pallas as pl
from jax.experimental.pallas import tpu as pltpu
from jax.experimental.pallas import tpu_sc as plsc

N_SH = 8
FEAT = 4096
BATCH = 1024
NC, NS, L = 2, 16, 16
NW = NC * NS
RPW = BATCH // NW
RC = 8
NRC = RPW // RC
JG = 1024
NJG = FEAT // JG
JBLK = JG // L


def _sc_shuffle(x_f, idx_g, w_g, bias):
    mesh = plsc.VectorSubcoreMesh(
        core_axis_name="c", subcore_axis_name="s",
        num_cores=NC, num_subcores=NS)

    @functools.partial(
        pl.kernel,
        out_type=jax.ShapeDtypeStruct((BATCH, FEAT), jnp.float32),
        mesh=mesh,
        scratch_types=[
            pltpu.VMEM((N_SH * JG,), jnp.int32),
            pltpu.VMEM((N_SH * JG,), jnp.float32),
            pltpu.VMEM((JG,), jnp.float32),
            pltpu.VMEM((RC * FEAT,), jnp.float32),
            pltpu.VMEM((RC * FEAT,), jnp.float32),
            pltpu.VMEM((RC, JG), jnp.float32),
            pltpu.VMEM((RC, JG), jnp.float32),
            [pltpu.SemaphoreType.DMA] * 2,
            [pltpu.SemaphoreType.DMA] * 2,
        ],
        compiler_params=pltpu.CompilerParams(needs_layout_passes=False),
    )
    def body(x_hbm, idx_hbm, w_hbm, b_hbm, out_hbm,
             idx_v, w_v, b_v, xr0_v, xr1_v, st0_v, st1_v, xsem, ssem):
        xr = [xr0_v, xr1_v]
        st = [st0_v, st1_v]
        wid = lax.axis_index("s") * NC + lax.axis_index("c")
        rbase = wid * RPW

        def start_xchunk(rc, b):
            pltpu.async_copy(
                x_hbm.at[pl.ds((rbase + rc * RC) * FEAT, RC * FEAT)],
                xr[b], xsem[b])

        start_xchunk(0, 0)

        for jg in range(NJG):
            pltpu.sync_copy(
                idx_hbm.at[pl.ds(jg * N_SH * JG, N_SH * JG)], idx_v)
            pltpu.sync_copy(
                w_hbm.at[pl.ds(jg * N_SH * JG, N_SH * JG)], w_v)
            pltpu.sync_copy(b_hbm.at[pl.ds(jg * JG, JG)], b_v)

            @pl.loop(0, NRC, step=2)
            def _rc(rc):
                for b in range(2):
                    rcc = rc + b
                    @pl.when(rcc + 1 < NRC)
                    def _():
                        start_xchunk(rcc + 1, 1 - b)
                    if jg < NJG - 1:
                        @pl.when(rcc + 1 == NRC)
                        def _():
                            start_xchunk(0, 1 - b)
                    pltpu.make_async_copy(
                        x_hbm.at[pl.ds(0, RC * FEAT)], xr[b],
                        xsem[b]).wait()
                    if jg == 0:
                        @pl.when(rcc >= 2)
                        def _():
                            pltpu.make_async_copy(
                                st[b],
                                out_hbm.at[pl.ds(rbase, RC), pl.ds(0, JG)],
                                ssem[b]).wait()
                    else:
                        pltpu.make_async_copy(
                            st[b],
                            out_hbm.at[pl.ds(rbase, RC), pl.ds(0, JG)],
                            ssem[b]).wait()

                    @plsc.parallel_loop(0, JBLK, unroll=2)
                    def _jb(jb):
                        c16 = jb * L
                        idxs = [idx_v[pl.ds(i * JG + c16, L)]
                                for i in range(N_SH)]
                        wvs = [w_v[pl.ds(i * JG + c16, L)]
                               for i in range(N_SH)]
                        bv = b_v[pl.ds(c16, L)]
                        for r in range(RC):
                            src = xr[b].at[pl.ds(r * FEAT, FEAT)]
                            acc = plsc.load_gather(src, [idxs[0]]) * wvs[0]
                            for i in range(1, N_SH):
                                acc = acc + (plsc.load_gather(src, [idxs[i]])
                                             * wvs[i])
                            st[b][r, pl.ds(c16, L)] = acc * 0.125 + bv

                    pltpu.async_copy(
                        st[b],
                        out_hbm.at[pl.ds(rbase + rcc * RC, RC),
                                   pl.ds(jg * JG, JG)],
                        ssem[b])

        for b in range(2):
            pltpu.make_async_copy(
                st[b], out_hbm.at[pl.ds(rbase, RC), pl.ds(0, JG)],
                ssem[b]).wait()

    return body(x_f, idx_g, w_g, bias)


def kernel(x, weights, bias, perms):
    x_f = x.reshape(-1)
    idx_g = perms.reshape(N_SH, NJG, JG).transpose(1, 0, 2).reshape(-1)
    w_g = weights.reshape(N_SH, NJG, JG).transpose(1, 0, 2).reshape(-1)
    return _sc_shuffle(x_f, idx_g, w_g, bias)


# speedup vs baseline: 1.1105x; 1.1105x over previous
import functools

import jax
import jax.numpy as jnp
from jax import lax
from jax.experimental import pallas as pl
from jax.experimental.pallas import tpu as pltpu
from jax.experimental.pallas import tpu_sc as plsc

N_SH = 8
FEAT = 4096
BATCH = 1024
NC, NS, L = 2, 16, 16
NW = NC * NS
RPW = BATCH // NW
RC = 8
NRC = RPW // RC
JG = 1024
NJG = FEAT // JG
JBLK = JG // L


def _sc_shuffle(x_f, idx_g, w_g, bias):
    mesh = plsc.VectorSubcoreMesh(
        core_axis_name="c", subcore_axis_name="s",
        num_cores=NC, num_subcores=NS)

    @functools.partial(
        pl.kernel,
        out_type=jax.ShapeDtypeStruct((BATCH, FEAT), jnp.float32),
        mesh=mesh,
        scratch_types=[
            pltpu.VMEM((N_SH * JG,), jnp.int32),
            pltpu.VMEM((N_SH * JG,), jnp.float32),
            pltpu.VMEM((JG,), jnp.float32),
            pltpu.VMEM((RC * FEAT,), jnp.float32),
            pltpu.VMEM((RC * FEAT,), jnp.float32),
            pltpu.VMEM((RC, JG), jnp.float32),
            pltpu.VMEM((RC, JG), jnp.float32),
            [pltpu.SemaphoreType.DMA] * 2,
            [pltpu.SemaphoreType.DMA] * 2,
        ],
        compiler_params=pltpu.CompilerParams(needs_layout_passes=False),
    )
    def body(x_hbm, idx_hbm, w_hbm, b_hbm, out_hbm,
             idx_v, w_v, b_v, xr0_v, xr1_v, st0_v, st1_v, xsem, ssem):
        xr = [xr0_v, xr1_v]
        st = [st0_v, st1_v]
        wid = lax.axis_index("s") * NC + lax.axis_index("c")
        rbase = wid * RPW

        def start_xchunk(rc, b):
            pltpu.async_copy(
                x_hbm.at[pl.ds((rbase + rc * RC) * FEAT, RC * FEAT)],
                xr[b], xsem[b])

        start_xchunk(0, 0)

        for jg in range(NJG):
            pltpu.sync_copy(
                idx_hbm.at[pl.ds(jg * N_SH * JG, N_SH * JG)], idx_v)
            pltpu.sync_copy(
                w_hbm.at[pl.ds(jg * N_SH * JG, N_SH * JG)], w_v)
            pltpu.sync_copy(b_hbm.at[pl.ds(jg * JG, JG)], b_v)

            @pl.loop(0, NRC, step=2)
            def _rc(rc):
                for b in range(2):
                    rcc = rc + b
                    @pl.when(rcc + 1 < NRC)
                    def _():
                        start_xchunk(rcc + 1, 1 - b)
                    if jg < NJG - 1:
                        @pl.when(rcc + 1 == NRC)
                        def _():
                            start_xchunk(0, 1 - b)
                    pltpu.make_async_copy(
                        x_hbm.at[pl.ds(0, RC * FEAT)], xr[b],
                        xsem[b]).wait()
                    if jg == 0:
                        @pl.when(rcc >= 2)
                        def _():
                            pltpu.make_async_copy(
                                st[b],
                                out_hbm.at[pl.ds(rbase, RC), pl.ds(0, JG)],
                                ssem[b]).wait()
                    else:
                        pltpu.make_async_copy(
                            st[b],
                            out_hbm.at[pl.ds(rbase, RC), pl.ds(0, JG)],
                            ssem[b]).wait()

                    @plsc.parallel_loop(0, JBLK, unroll=1)
                    def _jb(jb):
                        c16 = jb * L
                        idxs = [idx_v[pl.ds(i * JG + c16, L)]
                                for i in range(N_SH)]
                        wvs = [w_v[pl.ds(i * JG + c16, L)]
                               for i in range(N_SH)]
                        bv = b_v[pl.ds(c16, L)]
                        for r in range(RC):
                            src = xr[b].at[pl.ds(r * FEAT, FEAT)]
                            acc = plsc.load_gather(src, [idxs[0]]) * wvs[0]
                            for i in range(1, N_SH):
                                acc = acc + (plsc.load_gather(src, [idxs[i]])
                                             * wvs[i])
                            st[b][r, pl.ds(c16, L)] = acc * 0.125 + bv

                    pltpu.async_copy(
                        st[b],
                        out_hbm.at[pl.ds(rbase + rcc * RC, RC),
                                   pl.ds(jg * JG, JG)],
                        ssem[b])

        for b in range(2):
            pltpu.make_async_copy(
                st[b], out_hbm.at[pl.ds(rbase, RC), pl.ds(0, JG)],
                ssem[b]).wait()

    return body(x_f, idx_g, w_g, bias)


def kernel(x, weights, bias, perms):
    x_f = x.reshape(-1)
    idx_g = perms.reshape(N_SH, NJG, JG).transpose(1, 0, 2).reshape(-1)
    w_g = weights.reshape(N_SH, NJG, JG).transpose(1, 0, 2).reshape(-1)
    return _sc_shuffle(x_f, idx_g, w_g, bias)


# speedup vs baseline: 1.4280x; 1.2859x over previous
import functools

import jax
import jax.numpy as jnp
from jax import lax
from jax.experimental import pallas as pl
from jax.experimental.pallas import tpu as pltpu
from jax.experimental.pallas import tpu_sc as plsc

N_SH = 8
FEAT = 4096
BATCH = 1024
NC, NS, L = 2, 16, 16
NW = NC * NS
JPW = FEAT // NW
KJ = 2
NCHUNK = JPW // KJ
CVR = BATCH // L
NBUF = 4


def _sc_shuffle(xT, idx_flat, wsp, *, interpret=False):
    mesh = plsc.VectorSubcoreMesh(
        core_axis_name="c", subcore_axis_name="s",
        num_cores=NC, num_subcores=NS)

    GR = KJ * N_SH

    @functools.partial(
        pl.kernel,
        out_type=jax.ShapeDtypeStruct((FEAT, BATCH), jnp.float32),
        mesh=mesh,
        scratch_types=[
            pltpu.VMEM((JPW * N_SH,), jnp.int32),
            pltpu.VMEM((JPW * L,), jnp.float32),
            pltpu.VMEM((NBUF, GR, BATCH), jnp.float32),
            pltpu.VMEM((2, KJ, BATCH), jnp.float32),
            [pltpu.SemaphoreType.DMA] * NBUF,
            [pltpu.SemaphoreType.DMA] * 2,
        ],
        interpret=interpret,
    )
    def body(xT_hbm, idx_hbm, wsp_hbm, out_hbm,
             idx_v, wsp_v, rows_v, stage_v, gsem, ssem):
        wid = lax.axis_index("s") * NC + lax.axis_index("c")
        jbase = wid * JPW
        pltpu.sync_copy(idx_hbm.at[pl.ds(jbase * N_SH, JPW * N_SH)], idx_v)
        pltpu.sync_copy(wsp_hbm.at[pl.ds(jbase * L, JPW * L)], wsp_v)

        def start_gather(c, b):
            pltpu.async_copy(
                xT_hbm.at[idx_v.at[pl.ds(c * GR, GR)]], rows_v.at[b], gsem[b])

        start_gather(0, 0)
        start_gather(1, 1)

        @pl.loop(0, NCHUNK, step=NBUF)
        def _chunk(c):
            for b in range(NBUF):
                cc = c + b
                @pl.when(cc + 2 < NCHUNK)
                def _():
                    start_gather(cc + 2, (b + 2) % NBUF)
                pltpu.make_async_copy(
                    xT_hbm.at[pl.ds(0, GR)], rows_v.at[b], gsem[b]).wait()
                sb = b % 2
                @pl.when(cc >= 2)
                def _():
                    pltpu.make_async_copy(
                        stage_v.at[sb], out_hbm.at[pl.ds(jbase, KJ)],
                        ssem[sb]).wait()
                for jj in range(KJ):
                    jloc = cc * KJ + jj
                    wbv = wsp_v[pl.ds(jloc * L, L)]
                    ws = [wbv[i] * 0.125 for i in range(N_SH)]
                    bsc = wbv[N_SH]

                    @plsc.parallel_loop(0, CVR, unroll=8)
                    def _ch(ch):
                        acc = jnp.full((L,), bsc, jnp.float32)
                        for i in range(N_SH):
                            acc = acc + rows_v[b, jj * N_SH + i,
                                               pl.ds(ch * L, L)] * ws[i]
                        stage_v[sb, jj, pl.ds(ch * L, L)] = acc
                pltpu.async_copy(
                    stage_v.at[sb], out_hbm.at[pl.ds(jbase + cc * KJ, KJ)],
                    ssem[sb])

        for sb in range(2):
            pltpu.make_async_copy(
                stage_v.at[sb], out_hbm.at[pl.ds(jbase, KJ)], ssem[sb]).wait()

    return body(xT, idx_flat, wsp)


def kernel(x, weights, bias, perms):
    xT = x.T
    idx_flat = perms.T.reshape(-1)
    wsp = jnp.concatenate(
        [weights.T, bias[:, None],
         jnp.zeros((FEAT, L - N_SH - 1), jnp.float32)], axis=1).reshape(-1)
    outT = _sc_shuffle(xT, idx_flat, wsp)
    return outT.T
